# Initial kernel scaffold; baseline (speedup 1.0000x reference)
import jax
import jax.numpy as jnp
from jax.experimental import pallas as pl


def kernel(x, edge_index, batch_index, W1, b1, W2, b2, Wm, bm, Ws, bs):
    raise NotImplementedError("write your pallas kernel here")


# speedup vs baseline: 11.4861x; 11.4861x over previous
import functools

import jax
import jax.numpy as jnp
from jax import lax
from jax.experimental import pallas as pl
from jax.experimental.pallas import tpu as pltpu
from jax.experimental.pallas import tpu_sc as plsc

N = 10000
E = 320000
F = 128
H = 128
L = 64
G = 64

NC = 2
NS = 16
NW = NC * NS
CH = 128
ROWS_PER_TILE = 640
NPAD = NS * ROWS_PER_TILE
CHUNKS = -(-E // (NW * CH))
EPAD = NW * CH * CHUNKS
PER_W = CHUNKS * CH
NBLK = 16

_mesh = plsc.VectorSubcoreMesh(core_axis_name="c", subcore_axis_name="s")


@functools.partial(
    pl.kernel,
    out_type=jax.ShapeDtypeStruct((NC, NPAD, 16), jnp.float32),
    mesh=_mesh,
    scratch_types=[
        pltpu.VMEM_SHARED((NPAD, 16), jnp.float32),
        pltpu.VMEM((CHUNKS, CH), jnp.int32),
        pltpu.VMEM((CH, 16), jnp.float32),
    ],
)
def _sc_degree(dst3, zeros16, ones16, dp, acc, idx_v, ones_v):
    cid = lax.axis_index("c")
    sid = lax.axis_index("s")
    wid = sid * NC + cid
    row0 = sid * ROWS_PER_TILE
    pltpu.sync_copy(zeros16, acc.at[pl.ds(row0, ROWS_PER_TILE)])
    pltpu.sync_copy(dst3.at[wid], idx_v)
    pltpu.sync_copy(ones16, ones_v)
    plsc.subcore_barrier()

    def step(r, carry):
        pltpu.sync_copy(ones_v, acc.at[idx_v.at[r]], add=True)
        return carry

    lax.fori_loop(0, CHUNKS, step, 0)
    plsc.subcore_barrier()
    pltpu.sync_copy(acc.at[pl.ds(row0, ROWS_PER_TILE)],
                    dp.at[cid, pl.ds(row0, ROWS_PER_TILE)])


@functools.partial(
    pl.kernel,
    out_type=jax.ShapeDtypeStruct((NC, NPAD, H), jnp.float32),
    mesh=_mesh,
    scratch_types=[
        pltpu.VMEM_SHARED((NPAD, H), jnp.float32),
        pltpu.VMEM((CHUNKS, CH), jnp.int32),
        pltpu.VMEM((CHUNKS, CH), jnp.int32),
        pltpu.VMEM((CH, H), jnp.float32),
        pltpu.SemaphoreType.DMA,
    ],
)
def _sc_edge_agg(hs, src3, dst3, zerosH, ap, acc, sidx_v, didx_v, rows_v, sem):
    cid = lax.axis_index("c")
    sid = lax.axis_index("s")
    wid = sid * NC + cid
    row0 = sid * ROWS_PER_TILE
    pltpu.sync_copy(zerosH, acc.at[pl.ds(row0, ROWS_PER_TILE)])
    pltpu.sync_copy(src3.at[wid], sidx_v)
    pltpu.sync_copy(dst3.at[wid], didx_v)
    plsc.subcore_barrier()

    def step(r, carry):
        pltpu.async_copy(hs.at[sidx_v.at[r]], rows_v, sem).wait()
        pltpu.sync_copy(rows_v, acc.at[didx_v.at[r]], add=True)
        return carry

    lax.fori_loop(0, CHUNKS, step, 0)
    plsc.subcore_barrier()
    pltpu.sync_copy(acc.at[pl.ds(row0, ROWS_PER_TILE)],
                    ap.at[cid, pl.ds(row0, ROWS_PER_TILE)])


def _tc1_body(x_ref, dp0_ref, dp1_ref, w1_ref, hs_ref, dinv_ref):
    deg = dp0_ref[...] + dp1_ref[...] + 1.0
    dinv = 1.0 / jnp.sqrt(deg)
    dinv_ref[...] = dinv
    u = jnp.dot(x_ref[...], w1_ref[...], preferred_element_type=jnp.float32)
    hs_ref[...] = u * dinv[:, :1]


def _tc1(x_p, dp0, dp1, W1):
    blk = ROWS_PER_TILE
    return pl.pallas_call(
        _tc1_body,
        grid=(NBLK,),
        in_specs=[
            pl.BlockSpec((blk, F), lambda i: (i, 0)),
            pl.BlockSpec((blk, 16), lambda i: (i, 0)),
            pl.BlockSpec((blk, 16), lambda i: (i, 0)),
            pl.BlockSpec((F, H), lambda i: (0, 0)),
        ],
        out_specs=[
            pl.BlockSpec((blk, H), lambda i: (i, 0)),
            pl.BlockSpec((blk, 16), lambda i: (i, 0)),
        ],
        out_shape=[
            jax.ShapeDtypeStruct((NPAD, H), jnp.float32),
            jax.ShapeDtypeStruct((NPAD, 16), jnp.float32),
        ],
    )(x_p, dp0, dp1, W1)


def _tc2_body(a0_ref, a1_ref, hs1_ref, dinv_ref, w2_ref, b1_ref, hs2_ref):
    i = pl.program_id(0)
    dinv = dinv_ref[...][:, :1]
    t = (a0_ref[...] + a1_ref[...] + hs1_ref[...]) * dinv
    h = jnp.maximum(t + b1_ref[...], 0.0)
    u = jnp.dot(h, w2_ref[...], preferred_element_type=jnp.float32)
    rows = i * ROWS_PER_TILE + lax.broadcasted_iota(jnp.int32, (ROWS_PER_TILE, H), 0)
    hs2_ref[...] = jnp.where(rows < N, u * dinv, 0.0)


def _tc2(a0, a1, hs1, dinv, W2, b1r):
    blk = ROWS_PER_TILE
    return pl.pallas_call(
        _tc2_body,
        grid=(NBLK,),
        in_specs=[
            pl.BlockSpec((blk, H), lambda i: (i, 0)),
            pl.BlockSpec((blk, H), lambda i: (i, 0)),
            pl.BlockSpec((blk, H), lambda i: (i, 0)),
            pl.BlockSpec((blk, 16), lambda i: (i, 0)),
            pl.BlockSpec((H, H), lambda i: (0, 0)),
            pl.BlockSpec((1, H), lambda i: (0, 0)),
        ],
        out_specs=pl.BlockSpec((blk, H), lambda i: (i, 0)),
        out_shape=jax.ShapeDtypeStruct((NPAD, H), jnp.float32),
    )(a0, a1, hs1, dinv, W2, b1r)


def _tc3_body(a0_ref, a1_ref, hs2_ref, dinv_ref, b2_ref, batch_ref,
              wm_ref, bm_ref, ws_ref, bs_ref, mean_ref, std_ref,
              psum, cntm):
    i = pl.program_id(0)

    @pl.when(i == 0)
    def _init():
        psum[...] = jnp.zeros_like(psum)
        cntm[...] = jnp.zeros_like(cntm)

    dinv = dinv_ref[...][:, :1]
    t = (a0_ref[...] + a1_ref[...] + hs2_ref[...]) * dinv
    h = jnp.maximum(t + b2_ref[...], 0.0)
    rows = i * ROWS_PER_TILE + lax.broadcasted_iota(jnp.int32, (ROWS_PER_TILE, G), 0)
    gids = lax.broadcasted_iota(jnp.int32, (ROWS_PER_TILE, G), 1)
    onehot = jnp.where((batch_ref[...] == gids) & (rows < N), 1.0, 0.0)
    psum[...] += lax.dot_general(onehot, h, (((0,), (0,)), ((), ())),
                                 preferred_element_type=jnp.float32)
    ones = jnp.ones((ROWS_PER_TILE, H), jnp.float32)
    cntm[...] += lax.dot_general(onehot, ones, (((0,), (0,)), ((), ())),
                                 preferred_element_type=jnp.float32)

    @pl.when(i == NBLK - 1)
    def _final():
        p = psum[...] / jnp.maximum(cntm[...], 1.0)
        pm = p[:, :H]
        mean_ref[...] = jnp.dot(pm, wm_ref[...], preferred_element_type=jnp.float32) + bm_ref[...]
        std_ref[...] = jnp.exp(
            jnp.dot(pm, ws_ref[...], preferred_element_type=jnp.float32) + bs_ref[...])


def _tc3(a0, a1, hs2, dinv, b2r, batch_p, Wm, bmr, Ws, bsr):
    blk = ROWS_PER_TILE
    return pl.pallas_call(
        _tc3_body,
        grid=(NBLK,),
        in_specs=[
            pl.BlockSpec((blk, H), lambda i: (i, 0)),
            pl.BlockSpec((blk, H), lambda i: (i, 0)),
            pl.BlockSpec((blk, H), lambda i: (i, 0)),
            pl.BlockSpec((blk, 16), lambda i: (i, 0)),
            pl.BlockSpec((1, H), lambda i: (0, 0)),
            pl.BlockSpec((blk, 1), lambda i: (i, 0)),
            pl.BlockSpec((H, L), lambda i: (0, 0)),
            pl.BlockSpec((1, L), lambda i: (0, 0)),
            pl.BlockSpec((H, L), lambda i: (0, 0)),
            pl.BlockSpec((1, L), lambda i: (0, 0)),
        ],
        out_specs=[
            pl.BlockSpec((G, L), lambda i: (0, 0)),
            pl.BlockSpec((G, L), lambda i: (0, 0)),
        ],
        out_shape=[
            jax.ShapeDtypeStruct((G, L), jnp.float32),
            jax.ShapeDtypeStruct((G, L), jnp.float32),
        ],
        scratch_shapes=[
            pltpu.VMEM((G, H), jnp.float32),
            pltpu.VMEM((G, H), jnp.float32),
        ],
    )(a0, a1, hs2, dinv, b2r, batch_p, Wm, bmr, Ws, bsr)


def kernel(x, edge_index, batch_index, W1, b1, W2, b2, Wm, bm, Ws, bs):
    src = edge_index[0].astype(jnp.int32)
    dst = edge_index[1].astype(jnp.int32)
    src_p = jnp.concatenate(
        [src, jnp.full((EPAD - E,), N, jnp.int32)]).reshape(NW, CHUNKS, CH)
    dst_p = jnp.concatenate(
        [dst, jnp.zeros((EPAD - E,), jnp.int32)]).reshape(NW, CHUNKS, CH)
    x_p = jnp.zeros((NPAD, F), jnp.float32).at[:N].set(x)
    batch_p = jnp.pad(batch_index.astype(jnp.int32), (0, NPAD - N)).reshape(NPAD, 1)
    zeros16 = jnp.zeros((ROWS_PER_TILE, 16), jnp.float32)
    zerosH = jnp.zeros((ROWS_PER_TILE, H), jnp.float32)
    ones16 = jnp.ones((CH, 16), jnp.float32)
    b1r = b1.reshape(1, H)
    b2r = b2.reshape(1, H)
    bmr = bm.reshape(1, L)
    bsr = bs.reshape(1, L)

    dp = _sc_degree(dst_p, zeros16, ones16)
    hs1, dinv = _tc1(x_p, dp[0], dp[1], W1)
    ap1 = _sc_edge_agg(hs1, src_p, dst_p, zerosH)
    hs2 = _tc2(ap1[0], ap1[1], hs1, dinv, W2, b1r)
    ap2 = _sc_edge_agg(hs2, src_p, dst_p, zerosH)
    mean, std = _tc3(ap2[0], ap2[1], hs2, dinv, b2r, batch_p, Wm, bmr, Ws, bsr)
    return mean, std
